# Initial kernel scaffold; baseline (speedup 1.0000x reference)
import jax
import jax.numpy as jnp
from jax.experimental import pallas as pl


def kernel(x, edge_index, W1a, b1a, W1b, b1b, W2a, b2a, W2b, b2b):
    raise NotImplementedError("write your pallas kernel here")


# speedup vs baseline: 4.8633x; 4.8633x over previous
import functools

import jax
import jax.numpy as jnp
from jax import lax
from jax.experimental import pallas as pl
from jax.experimental.pallas import tpu as pltpu
from jax.experimental.pallas import tpu_sc as plsc

_NC, _NS = 2, 16
_NW = _NC * _NS
_CH = 80


def _sc_aggregate(x, src, dst, zeros):
    N, D = x.shape
    E = src.shape[0]
    npad = ((N + 8 * _NS - 1) // (8 * _NS)) * (8 * _NS)
    epw = E // _NW
    nch = epw // _CH
    rpt = npad // _NS
    zr = zeros.shape[0]
    nz = rpt // zr
    mesh = plsc.VectorSubcoreMesh(core_axis_name="c", subcore_axis_name="s")

    @functools.partial(
        pl.kernel,
        out_type=jax.ShapeDtypeStruct((_NC, npad, D), jnp.float32),
        mesh=mesh,
        scratch_types=[
            pltpu.VMEM((_CH,), jnp.int32),
            pltpu.VMEM((_CH,), jnp.int32),
            pltpu.VMEM((_CH, D), jnp.float32),
            pltpu.VMEM_SHARED((npad, D), jnp.float32),
            pltpu.SemaphoreType.DMA,
        ],
    )
    def agg(x_hbm, src_hbm, dst_hbm, z_hbm, out_hbm, src_v, dst_v, rows_v, acc, sem):
        c = lax.axis_index("c")
        s = lax.axis_index("s")
        row0 = s * rpt
        for k in range(nz):
            pltpu.sync_copy(z_hbm, acc.at[pl.ds(row0 + k * zr, zr)])
        plsc.subcore_barrier()

        base_e = (c * _NS + s) * epw

        def body(g, carry):
            off = base_e + g * _CH
            pltpu.sync_copy(src_hbm.at[pl.ds(off, _CH)], src_v)
            pltpu.async_copy(x_hbm.at[src_v], rows_v, sem).wait()
            pltpu.sync_copy(dst_hbm.at[pl.ds(off, _CH)], dst_v)
            pltpu.sync_copy(rows_v, acc.at[dst_v], add=True)
            return carry

        lax.fori_loop(0, nch, body, 0)
        plsc.subcore_barrier()
        pltpu.sync_copy(acc.at[pl.ds(row0, rpt)], out_hbm.at[c, pl.ds(row0, rpt)])

    return agg(x, src, dst, zeros)[:, :N, :]


def _tc_mlp(x, p0, p1, Wa, ba, Wb, bb):
    N, D = x.shape
    H = Wa.shape[1]
    br = 400
    grid = (N // br,)

    def body(x_ref, p0_ref, p1_ref, wa_ref, ba_ref, wb_ref, bb_ref, o_ref):
        h = x_ref[...] + p0_ref[...] + p1_ref[...]
        h = jnp.dot(h, wa_ref[...], preferred_element_type=jnp.float32) + ba_ref[...]
        h = jnp.maximum(h, 0.0)
        h = jnp.dot(h, wb_ref[...], preferred_element_type=jnp.float32) + bb_ref[...]
        o_ref[...] = jnp.maximum(h, 0.0)

    return pl.pallas_call(
        body,
        grid=grid,
        in_specs=[
            pl.BlockSpec((br, D), lambda i: (i, 0)),
            pl.BlockSpec((br, D), lambda i: (i, 0)),
            pl.BlockSpec((br, D), lambda i: (i, 0)),
            pl.BlockSpec((D, H), lambda i: (0, 0)),
            pl.BlockSpec((1, H), lambda i: (0, 0)),
            pl.BlockSpec((H, H), lambda i: (0, 0)),
            pl.BlockSpec((1, H), lambda i: (0, 0)),
        ],
        out_specs=pl.BlockSpec((br, H), lambda i: (i, 0)),
        out_shape=jax.ShapeDtypeStruct((N, H), jnp.float32),
    )(x, p0, p1, Wa, ba.reshape(1, -1), Wb, bb.reshape(1, -1))


def kernel(x, edge_index, W1a, b1a, W1b, b1b, W2a, b2a, W2b, b2b):
    ei = edge_index.astype(jnp.int32)
    src, dst = ei[0], ei[1]
    zeros = jnp.zeros((128, x.shape[1]), jnp.float32)
    parts = _sc_aggregate(x, src, dst, zeros)
    h = _tc_mlp(x, parts[0], parts[1], W1a, b1a, W1b, b1b)
    parts2 = _sc_aggregate(h, src, dst, zeros)
    return _tc_mlp(h, parts2[0], parts2[1], W2a, b2a, W2b, b2b)


# speedup vs baseline: 6.8119x; 1.4007x over previous
import functools

import jax
import jax.numpy as jnp
from jax import lax
from jax.experimental import pallas as pl
from jax.experimental.pallas import tpu as pltpu
from jax.experimental.pallas import tpu_sc as plsc

_NC, _NS = 2, 16
_NW = _NC * _NS
_CH = 80
_NBUF = 2


def _sc_aggregate(x, src, dst, zeros):
    N, D = x.shape
    E = src.shape[0]
    ch = _CH
    epw = E // _NW
    nch = epw // ch
    npad = ((N + 8 * _NS - 1) // (8 * _NS)) * (8 * _NS)
    rpt = npad // _NS
    zr = zeros.shape[0]
    nz = rpt // zr
    mesh = plsc.VectorSubcoreMesh(core_axis_name="c", subcore_axis_name="s")

    @functools.partial(
        pl.kernel,
        out_type=jax.ShapeDtypeStruct((_NC, npad, D), jnp.float32),
        mesh=mesh,
        scratch_types=[
            [pltpu.VMEM((ch,), jnp.int32) for _ in range(_NBUF)],
            [pltpu.VMEM((ch,), jnp.int32) for _ in range(_NBUF)],
            [pltpu.VMEM((ch, D), jnp.float32) for _ in range(_NBUF)],
            pltpu.VMEM_SHARED((npad, D), jnp.float32),
            pltpu.SemaphoreType.DMA,
            [pltpu.SemaphoreType.DMA for _ in range(_NBUF)],
        ],
    )
    def agg(x_hbm, src_hbm, dst_hbm, z_hbm, out_hbm, sidx, didx, rows, acc,
            gsem, isem):
        c = lax.axis_index("c")
        s = lax.axis_index("s")
        w = c * _NS + s
        row0 = s * rpt
        base_e = w * epw

        def issue_idx(g, b):
            off = base_e + g * ch
            pltpu.async_copy(src_hbm.at[pl.ds(off, ch)], sidx[b], isem[b])
            pltpu.async_copy(dst_hbm.at[pl.ds(off, ch)], didx[b], isem[b])

        def wait_idx(b):
            pltpu.make_async_copy(src_hbm.at[pl.ds(0, ch)], sidx[b], isem[b]).wait()
            pltpu.make_async_copy(dst_hbm.at[pl.ds(0, ch)], didx[b], isem[b]).wait()

        def scatter(b):
            pltpu.sync_copy(rows[b], acc.at[didx[b]], add=True)

        issue_idx(0, 0)
        issue_idx(1, 1)
        for k in range(nz):
            pltpu.sync_copy(z_hbm, acc.at[pl.ds(row0 + k * zr, zr)])
        wait_idx(0)
        pltpu.async_copy(x_hbm.at[sidx[0]], rows[0], gsem).wait()
        plsc.subcore_barrier()

        def step(g, b):
            b1 = (b + 1) % _NBUF

            @pl.when(g + 1 < nch)
            def _steady():
                wait_idx(b1)
                d = pltpu.async_copy(x_hbm.at[sidx[b1]], rows[b1], gsem)
                scatter(b)
                d.wait()

            @pl.when(g + 1 >= nch)
            def _last():
                scatter(b)

            @pl.when(g + 2 < nch)
            def _next_idx():
                issue_idx(g + 2, b)

        def outer(k, carry):
            for b in range(_NBUF):
                step(k * _NBUF + b, b)
            return carry

        lax.fori_loop(0, nch // _NBUF, outer, 0)
        for g in range((nch // _NBUF) * _NBUF, nch):
            step(g, g % _NBUF)
        plsc.subcore_barrier()
        pltpu.sync_copy(acc.at[pl.ds(row0, rpt)], out_hbm.at[c, pl.ds(row0, rpt)])

    return agg(x, src, dst, zeros)[:, :N, :]


def _tc_mlp(x, p0, p1, Wa, ba, Wb, bb):
    N, D = x.shape
    H = Wa.shape[1]
    br = 400
    grid = (N // br,)

    def body(x_ref, p0_ref, p1_ref, wa_ref, ba_ref, wb_ref, bb_ref, o_ref):
        h = x_ref[...] + p0_ref[...] + p1_ref[...]
        h = jnp.dot(h, wa_ref[...], preferred_element_type=jnp.float32) + ba_ref[...]
        h = jnp.maximum(h, 0.0)
        h = jnp.dot(h, wb_ref[...], preferred_element_type=jnp.float32) + bb_ref[...]
        o_ref[...] = jnp.maximum(h, 0.0)

    return pl.pallas_call(
        body,
        grid=grid,
        in_specs=[
            pl.BlockSpec((br, D), lambda i: (i, 0)),
            pl.BlockSpec((br, D), lambda i: (i, 0)),
            pl.BlockSpec((br, D), lambda i: (i, 0)),
            pl.BlockSpec((D, H), lambda i: (0, 0)),
            pl.BlockSpec((1, H), lambda i: (0, 0)),
            pl.BlockSpec((H, H), lambda i: (0, 0)),
            pl.BlockSpec((1, H), lambda i: (0, 0)),
        ],
        out_specs=pl.BlockSpec((br, H), lambda i: (i, 0)),
        out_shape=jax.ShapeDtypeStruct((N, H), jnp.float32),
    )(x, p0, p1, Wa, ba.reshape(1, -1), Wb, bb.reshape(1, -1))


def kernel(x, edge_index, W1a, b1a, W1b, b1b, W2a, b2a, W2b, b2b):
    ei = edge_index.astype(jnp.int32)
    src, dst = ei[0], ei[1]
    zeros = jnp.zeros((128, x.shape[1]), jnp.float32)
    parts = _sc_aggregate(x, src, dst, zeros)
    h = _tc_mlp(x, parts[0], parts[1], W1a, b1a, W1b, b1b)
    parts2 = _sc_aggregate(h, src, dst, zeros)
    return _tc_mlp(h, parts2[0], parts2[1], W2a, b2a, W2b, b2b)


# speedup vs baseline: 8.2021x; 1.2041x over previous
import functools

import jax
import jax.numpy as jnp
from jax import lax
from jax.experimental import pallas as pl
from jax.experimental.pallas import tpu as pltpu
from jax.experimental.pallas import tpu_sc as plsc

_NC, _NS = 2, 16
_NW = _NC * _NS
_CH = 80
_NB = 3
_BLK = 6


def _sc_aggregate(x, src, dst, zeros):
    N, D = x.shape
    E = src.shape[0]
    ch = _CH
    epw = E // _NW
    nch = epw // ch
    nblk = nch // _BLK
    tail = nch - nblk * _BLK
    npad = ((N + 8 * _NS - 1) // (8 * _NS)) * (8 * _NS)
    rpt = npad // _NS
    assert zeros.shape[0] == rpt
    mesh = plsc.VectorSubcoreMesh(core_axis_name="c", subcore_axis_name="s")

    @functools.partial(
        pl.kernel,
        out_type=[jax.ShapeDtypeStruct((npad, D), jnp.float32),
                  jax.ShapeDtypeStruct((npad, D), jnp.float32)],
        mesh=mesh,
        scratch_types=[
            pltpu.VMEM((epw,), jnp.int32),
            [pltpu.VMEM((ch,), jnp.int32) for _ in range(_BLK)],
            [pltpu.VMEM((ch, D), jnp.float32) for _ in range(_NB)],
            pltpu.VMEM_SHARED((npad, D), jnp.float32),
            [pltpu.SemaphoreType.DMA for _ in range(_NB)],
            [pltpu.SemaphoreType.DMA for _ in range(_NB)],
            [pltpu.SemaphoreType.DMA for _ in range(_BLK)],
        ],
    )
    def agg(x_hbm, src_hbm, dst_hbm, z_hbm, out0_hbm, out1_hbm, sidx, didx,
            rows, acc, gsem, ssem, isem):
        c = lax.axis_index("c")
        s = lax.axis_index("s")
        w = c * _NS + s
        row0 = s * rpt
        base_e = w * epw

        def issue_didx(g, i):
            pltpu.async_copy(dst_hbm.at[pl.ds(base_e + g * ch, ch)], didx[i],
                             isem[i])

        def wait_didx(i):
            pltpu.make_async_copy(dst_hbm.at[pl.ds(0, ch)], didx[i],
                                  isem[i]).wait()

        def gather_src(g):
            return x_hbm.at[sidx.at[pl.ds(g * ch, ch)]]

        def run_block(chunks):
            L = len(chunks)
            dgs = [None] * L
            dss = [None] * L
            dgs[0] = pltpu.async_copy(gather_src(chunks[0]), rows[0], gsem[0])
            for n in range(L):
                r = n % _NB
                dgs[n].wait()
                dss[n] = pltpu.async_copy(rows[r], acc.at[didx[n]], ssem[r],
                                          add=True)
                if n + 1 < L:
                    if n - 2 >= 0:
                        dss[n - 2].wait()
                    r1 = (n + 1) % _NB
                    dgs[n + 1] = pltpu.async_copy(gather_src(chunks[n + 1]),
                                                  rows[r1], gsem[r1])
            for m in range(max(0, L - 3), L):
                dss[m].wait()

        pltpu.sync_copy(src_hbm.at[pl.ds(base_e, epw)], sidx)
        for j in range(_BLK):
            issue_didx(j, j)
        pltpu.sync_copy(z_hbm, acc.at[pl.ds(row0, rpt)])
        plsc.subcore_barrier()

        def outer(k, carry):
            base = k * _BLK
            for j in range(_BLK):
                wait_didx(j)
            run_block([base + j for j in range(_BLK)])
            for j in range(_BLK):
                @pl.when(base + _BLK + j < nch)
                def _prefetch():
                    issue_didx(base + _BLK + j, j)
            return carry

        lax.fori_loop(0, nblk, outer, 0)
        if tail:
            for j in range(tail):
                wait_didx(j)
            run_block([nblk * _BLK + j for j in range(tail)])
        plsc.subcore_barrier()

        @pl.when(c == 0)
        def _w0():
            pltpu.sync_copy(acc.at[pl.ds(row0, rpt)], out0_hbm.at[pl.ds(row0, rpt)])

        @pl.when(c == 1)
        def _w1():
            pltpu.sync_copy(acc.at[pl.ds(row0, rpt)], out1_hbm.at[pl.ds(row0, rpt)])

    return agg(x, src, dst, zeros)


def _tc_mlp(x, p0, p1, Wa, ba, Wb, bb):
    N, D = x.shape
    Np = p0.shape[0]
    H = Wa.shape[1]
    br = 400
    grid = (N // br,)

    def body(x_ref, p0_ref, p1_ref, wa_ref, ba_ref, wb_ref, bb_ref, o_ref):
        h = x_ref[...] + p0_ref[...] + p1_ref[...]
        h = jnp.dot(h, wa_ref[...], preferred_element_type=jnp.float32) + ba_ref[...]
        h = jnp.maximum(h, 0.0)
        h = jnp.dot(h, wb_ref[...], preferred_element_type=jnp.float32) + bb_ref[...]
        o_ref[...] = jnp.maximum(h, 0.0)

    return pl.pallas_call(
        body,
        grid=grid,
        in_specs=[
            pl.BlockSpec((br, D), lambda i: (i, 0)),
            pl.BlockSpec((br, D), lambda i: (i, 0)),
            pl.BlockSpec((br, D), lambda i: (i, 0)),
            pl.BlockSpec((D, H), lambda i: (0, 0)),
            pl.BlockSpec((1, H), lambda i: (0, 0)),
            pl.BlockSpec((H, H), lambda i: (0, 0)),
            pl.BlockSpec((1, H), lambda i: (0, 0)),
        ],
        out_specs=pl.BlockSpec((br, H), lambda i: (i, 0)),
        out_shape=jax.ShapeDtypeStruct((N, H), jnp.float32),
    )(x, p0, p1, Wa, ba.reshape(1, -1), Wb, bb.reshape(1, -1))


def kernel(x, edge_index, W1a, b1a, W1b, b1b, W2a, b2a, W2b, b2b):
    ei = edge_index.astype(jnp.int32)
    src, dst = ei[0], ei[1]
    N, D = x.shape
    npad = ((N + 8 * _NS - 1) // (8 * _NS)) * (8 * _NS)
    zeros = jnp.zeros((npad // _NS, D), jnp.float32)
    p0, p1 = _sc_aggregate(x, src, dst, zeros)
    h = _tc_mlp(x, p0, p1, W1a, b1a, W1b, b1b)
    q0, q1 = _sc_aggregate(h, src, dst, zeros)
    return _tc_mlp(h, q0, q1, W2a, b2a, W2b, b2b)


# speedup vs baseline: 8.2053x; 1.0004x over previous
import functools

import jax
import jax.numpy as jnp
from jax import lax
from jax.experimental import pallas as pl
from jax.experimental.pallas import tpu as pltpu
from jax.experimental.pallas import tpu_sc as plsc

_NC, _NS = 2, 16
_NW = _NC * _NS
_CH = 80
_NB = 4
_BLK = 6


def _sc_aggregate(x, src, dst, zeros):
    N, D = x.shape
    E = src.shape[0]
    ch = _CH
    epw = E // _NW
    nch = epw // ch
    nblk = nch // _BLK
    tail = nch - nblk * _BLK
    npad = ((N + 8 * _NS - 1) // (8 * _NS)) * (8 * _NS)
    rpt = npad // _NS
    assert zeros.shape[0] == rpt
    mesh = plsc.VectorSubcoreMesh(core_axis_name="c", subcore_axis_name="s")

    @functools.partial(
        pl.kernel,
        out_type=[jax.ShapeDtypeStruct((npad, D), jnp.float32),
                  jax.ShapeDtypeStruct((npad, D), jnp.float32)],
        mesh=mesh,
        scratch_types=[
            [pltpu.VMEM((ch,), jnp.int32) for _ in range(_BLK)],
            [pltpu.VMEM((ch,), jnp.int32) for _ in range(_BLK)],
            [pltpu.VMEM((ch, D), jnp.float32) for _ in range(_NB)],
            pltpu.VMEM_SHARED((npad, D), jnp.float32),
            [pltpu.SemaphoreType.DMA for _ in range(_NB)],
            [pltpu.SemaphoreType.DMA for _ in range(_NB)],
            [pltpu.SemaphoreType.DMA for _ in range(_BLK)],
        ],
    )
    def agg(x_hbm, src_hbm, dst_hbm, z_hbm, out0_hbm, out1_hbm, sidx, didx,
            rows, acc, gsem, ssem, isem):
        c = lax.axis_index("c")
        s = lax.axis_index("s")
        w = c * _NS + s
        row0 = s * rpt
        base_e = w * epw

        def issue_idx(g, i):
            off = base_e + g * ch
            pltpu.async_copy(src_hbm.at[pl.ds(off, ch)], sidx[i], isem[i])
            pltpu.async_copy(dst_hbm.at[pl.ds(off, ch)], didx[i], isem[i])

        def wait_idx(i):
            pltpu.make_async_copy(src_hbm.at[pl.ds(0, ch)], sidx[i], isem[i]).wait()
            pltpu.make_async_copy(dst_hbm.at[pl.ds(0, ch)], didx[i], isem[i]).wait()

        def run_block(L):
            dgs = [None] * L
            dss = [None] * L
            dgs[0] = pltpu.async_copy(x_hbm.at[sidx[0]], rows[0], gsem[0])
            for n in range(L):
                r = n % _NB
                dgs[n].wait()
                dss[n] = pltpu.async_copy(rows[r], acc.at[didx[n]], ssem[r],
                                          add=True)
                if n + 1 < L:
                    if n - (_NB - 1) >= 0:
                        dss[n - (_NB - 1)].wait()
                    r1 = (n + 1) % _NB
                    dgs[n + 1] = pltpu.async_copy(x_hbm.at[sidx[n + 1]],
                                                  rows[r1], gsem[r1])
            for m in range(max(0, L - _NB), L):
                dss[m].wait()

        for j in range(_BLK):
            issue_idx(j, j)
        pltpu.sync_copy(z_hbm, acc.at[pl.ds(row0, rpt)])
        plsc.subcore_barrier()

        def outer(k, carry):
            base = k * _BLK
            for j in range(_BLK):
                wait_idx(j)
            run_block(_BLK)
            for j in range(_BLK):
                @pl.when(base + _BLK + j < nch)
                def _prefetch():
                    issue_idx(base + _BLK + j, j)
            return carry

        lax.fori_loop(0, nblk, outer, 0)
        if tail:
            for j in range(tail):
                wait_idx(j)
            run_block(tail)
        plsc.subcore_barrier()

        @pl.when(c == 0)
        def _w0():
            pltpu.sync_copy(acc.at[pl.ds(row0, rpt)], out0_hbm.at[pl.ds(row0, rpt)])

        @pl.when(c == 1)
        def _w1():
            pltpu.sync_copy(acc.at[pl.ds(row0, rpt)], out1_hbm.at[pl.ds(row0, rpt)])

    return agg(x, src, dst, zeros)


def _tc_mlp(x, p0, p1, Wa, ba, Wb, bb):
    N, D = x.shape
    Np = p0.shape[0]
    H = Wa.shape[1]
    br = 400
    grid = (N // br,)

    def body(x_ref, p0_ref, p1_ref, wa_ref, ba_ref, wb_ref, bb_ref, o_ref):
        h = x_ref[...] + p0_ref[...] + p1_ref[...]
        h = jnp.dot(h, wa_ref[...], preferred_element_type=jnp.float32) + ba_ref[...]
        h = jnp.maximum(h, 0.0)
        h = jnp.dot(h, wb_ref[...], preferred_element_type=jnp.float32) + bb_ref[...]
        o_ref[...] = jnp.maximum(h, 0.0)

    return pl.pallas_call(
        body,
        grid=grid,
        in_specs=[
            pl.BlockSpec((br, D), lambda i: (i, 0)),
            pl.BlockSpec((br, D), lambda i: (i, 0)),
            pl.BlockSpec((br, D), lambda i: (i, 0)),
            pl.BlockSpec((D, H), lambda i: (0, 0)),
            pl.BlockSpec((1, H), lambda i: (0, 0)),
            pl.BlockSpec((H, H), lambda i: (0, 0)),
            pl.BlockSpec((1, H), lambda i: (0, 0)),
        ],
        out_specs=pl.BlockSpec((br, H), lambda i: (i, 0)),
        out_shape=jax.ShapeDtypeStruct((N, H), jnp.float32),
    )(x, p0, p1, Wa, ba.reshape(1, -1), Wb, bb.reshape(1, -1))


def kernel(x, edge_index, W1a, b1a, W1b, b1b, W2a, b2a, W2b, b2b):
    ei = edge_index.astype(jnp.int32)
    src, dst = ei[0], ei[1]
    N, D = x.shape
    npad = ((N + 8 * _NS - 1) // (8 * _NS)) * (8 * _NS)
    zeros = jnp.zeros((npad // _NS, D), jnp.float32)
    p0, p1 = _sc_aggregate(x, src, dst, zeros)
    h = _tc_mlp(x, p0, p1, W1a, b1a, W1b, b1b)
    q0, q1 = _sc_aggregate(h, src, dst, zeros)
    return _tc_mlp(h, q0, q1, W2a, b2a, W2b, b2b)


# speedup vs baseline: 8.5209x; 1.0385x over previous
import functools

import jax
import jax.numpy as jnp
from jax import lax
from jax.experimental import pallas as pl
from jax.experimental.pallas import tpu as pltpu
from jax.experimental.pallas import tpu_sc as plsc

_NC, _NS = 2, 16
_NW = _NC * _NS
_CH = 80
_NB = 4
_BLK = 6


def _sc_aggregate(x, src, dst, zeros):
    N, D = x.shape
    E = src.shape[0]
    ch = _CH
    epw = E // _NW
    nch = epw // ch
    nblk = nch // _BLK
    tail = nch - nblk * _BLK
    npad = ((N + 8 * _NS - 1) // (8 * _NS)) * (8 * _NS)
    rpt = npad // _NS
    assert zeros.shape[0] == rpt
    mesh = plsc.VectorSubcoreMesh(core_axis_name="c", subcore_axis_name="s")

    @functools.partial(
        pl.kernel,
        out_type=[jax.ShapeDtypeStruct((npad, D), jnp.float32),
                  jax.ShapeDtypeStruct((npad, D), jnp.float32)],
        mesh=mesh,
        scratch_types=[
            [pltpu.VMEM((ch,), jnp.int32) for _ in range(_BLK)],
            [pltpu.VMEM((ch,), jnp.int32) for _ in range(_BLK)],
            [pltpu.VMEM((ch, D), jnp.float32) for _ in range(_NB)],
            pltpu.VMEM_SHARED((npad, D), jnp.float32),
            [pltpu.SemaphoreType.DMA for _ in range(_NB)],
            [pltpu.SemaphoreType.DMA for _ in range(_NB)],
            [pltpu.SemaphoreType.DMA for _ in range(_BLK)],
        ],
    )
    def agg(x_hbm, src_hbm, dst_hbm, z_hbm, out0_hbm, out1_hbm, sidx, didx,
            rows, acc, gsem, ssem, isem):
        c = lax.axis_index("c")
        s = lax.axis_index("s")
        w = c * _NS + s
        row0 = s * rpt
        base_e = w * epw

        def issue_idx(g, i):
            off = base_e + g * ch
            pltpu.async_copy(src_hbm.at[pl.ds(off, ch)], sidx[i], isem[i])
            pltpu.async_copy(dst_hbm.at[pl.ds(off, ch)], didx[i], isem[i])

        def wait_idx(i):
            pltpu.make_async_copy(src_hbm.at[pl.ds(0, ch)], sidx[i], isem[i]).wait()
            pltpu.make_async_copy(dst_hbm.at[pl.ds(0, ch)], didx[i], isem[i]).wait()

        def run_block(L):
            dgs = [None] * L
            dss = [None] * L
            dgs[0] = pltpu.async_copy(x_hbm.at[sidx[0]], rows[0], gsem[0])
            for n in range(L):
                r = n % _NB
                dgs[n].wait()
                dss[n] = pltpu.async_copy(rows[r], acc.at[didx[n]], ssem[r],
                                          add=True)
                if n + 1 < L:
                    if n - (_NB - 1) >= 0:
                        dss[n - (_NB - 1)].wait()
                    r1 = (n + 1) % _NB
                    dgs[n + 1] = pltpu.async_copy(x_hbm.at[sidx[n + 1]],
                                                  rows[r1], gsem[r1])
            for m in range(max(0, L - _NB), L):
                dss[m].wait()

        for j in range(_BLK):
            issue_idx(j, j)
        pltpu.sync_copy(z_hbm, acc.at[pl.ds(row0, rpt)])
        plsc.subcore_barrier()

        def outer(k, carry):
            base = k * _BLK
            for j in range(_BLK):
                wait_idx(j)
            run_block(_BLK)
            for j in range(_BLK):
                @pl.when(base + _BLK + j < nch)
                def _prefetch():
                    issue_idx(base + _BLK + j, j)
            return carry

        lax.fori_loop(0, nblk, outer, 0)
        if tail:
            for j in range(tail):
                wait_idx(j)
            run_block(tail)
        plsc.subcore_barrier()

        @pl.when(c == 0)
        def _w0():
            pltpu.sync_copy(acc.at[pl.ds(row0, rpt)], out0_hbm.at[pl.ds(row0, rpt)])

        @pl.when(c == 1)
        def _w1():
            pltpu.sync_copy(acc.at[pl.ds(row0, rpt)], out1_hbm.at[pl.ds(row0, rpt)])

    return agg(x, src, dst, zeros)


def _tc_mlp(x, p0, p1, Wa, ba, Wb, bb):
    N, D = x.shape
    Np = p0.shape[0]
    H = Wa.shape[1]
    br = 1000
    grid = (N // br,)

    def body(x_ref, p0_ref, p1_ref, wa_ref, ba_ref, wb_ref, bb_ref, o_ref):
        h = x_ref[...] + p0_ref[...] + p1_ref[...]
        h = jnp.dot(h, wa_ref[...], preferred_element_type=jnp.float32) + ba_ref[...]
        h = jnp.maximum(h, 0.0)
        h = jnp.dot(h, wb_ref[...], preferred_element_type=jnp.float32) + bb_ref[...]
        o_ref[...] = jnp.maximum(h, 0.0)

    return pl.pallas_call(
        body,
        grid=grid,
        in_specs=[
            pl.BlockSpec((br, D), lambda i: (i, 0)),
            pl.BlockSpec((br, D), lambda i: (i, 0)),
            pl.BlockSpec((br, D), lambda i: (i, 0)),
            pl.BlockSpec((D, H), lambda i: (0, 0)),
            pl.BlockSpec((1, H), lambda i: (0, 0)),
            pl.BlockSpec((H, H), lambda i: (0, 0)),
            pl.BlockSpec((1, H), lambda i: (0, 0)),
        ],
        out_specs=pl.BlockSpec((br, H), lambda i: (i, 0)),
        out_shape=jax.ShapeDtypeStruct((N, H), jnp.float32),
    )(x, p0, p1, Wa, ba.reshape(1, -1), Wb, bb.reshape(1, -1))


def kernel(x, edge_index, W1a, b1a, W1b, b1b, W2a, b2a, W2b, b2b):
    ei = edge_index.astype(jnp.int32)
    src, dst = ei[0], ei[1]
    N, D = x.shape
    npad = ((N + 8 * _NS - 1) // (8 * _NS)) * (8 * _NS)
    zeros = jnp.zeros((npad // _NS, D), jnp.float32)
    p0, p1 = _sc_aggregate(x, src, dst, zeros)
    h = _tc_mlp(x, p0, p1, W1a, b1a, W1b, b1b)
    q0, q1 = _sc_aggregate(h, src, dst, zeros)
    return _tc_mlp(h, q0, q1, W2a, b2a, W2b, b2b)


# speedup vs baseline: 8.6577x; 1.0161x over previous
import functools

import jax
import jax.numpy as jnp
from jax import lax
from jax.experimental import pallas as pl
from jax.experimental.pallas import tpu as pltpu
from jax.experimental.pallas import tpu_sc as plsc

_NC, _NS = 2, 16
_NW = _NC * _NS
_CH = 80
_NB = 4
_BLK = 6


def _sc_aggregate(x, src, dst, zeros):
    N, D = x.shape
    E = src.shape[0]
    ch = _CH
    epw = E // _NW
    nch = epw // ch
    nblk = nch // _BLK
    tail = nch - nblk * _BLK
    npad = ((N + 8 * _NS - 1) // (8 * _NS)) * (8 * _NS)
    rpt = npad // _NS
    assert zeros.shape[0] == rpt
    mesh = plsc.VectorSubcoreMesh(core_axis_name="c", subcore_axis_name="s")

    @functools.partial(
        pl.kernel,
        out_type=[jax.ShapeDtypeStruct((npad, D), jnp.float32),
                  jax.ShapeDtypeStruct((npad, D), jnp.float32)],
        mesh=mesh,
        scratch_types=[
            [pltpu.VMEM((ch,), jnp.int32) for _ in range(_BLK)],
            [pltpu.VMEM((ch,), jnp.int32) for _ in range(_BLK)],
            [pltpu.VMEM((ch, D), jnp.float32) for _ in range(_NB)],
            pltpu.VMEM_SHARED((npad, D), jnp.float32),
            [pltpu.SemaphoreType.DMA for _ in range(_NB)],
            [pltpu.SemaphoreType.DMA for _ in range(_NB)],
            [pltpu.SemaphoreType.DMA for _ in range(_BLK)],
        ],
    )
    def agg(x_hbm, src_hbm, dst_hbm, z_hbm, out0_hbm, out1_hbm, sidx, didx,
            rows, acc, gsem, ssem, isem):
        c = lax.axis_index("c")
        s = lax.axis_index("s")
        w = c * _NS + s
        row0 = s * rpt
        base_e = w * epw

        def issue_idx(g, i):
            off = base_e + g * ch
            pltpu.async_copy(src_hbm.at[pl.ds(off, ch)], sidx[i], isem[i])
            pltpu.async_copy(dst_hbm.at[pl.ds(off, ch)], didx[i], isem[i])

        def wait_idx(i):
            pltpu.make_async_copy(src_hbm.at[pl.ds(0, ch)], sidx[i], isem[i]).wait()
            pltpu.make_async_copy(dst_hbm.at[pl.ds(0, ch)], didx[i], isem[i]).wait()

        def run_block(L):
            dgs = [None] * L
            dss = [None] * L
            dgs[0] = pltpu.async_copy(x_hbm.at[sidx[0]], rows[0], gsem[0])
            for n in range(L):
                r = n % _NB
                dgs[n].wait()
                dss[n] = pltpu.async_copy(rows[r], acc.at[didx[n]], ssem[r],
                                          add=True)
                if n + 1 < L:
                    if n - (_NB - 1) >= 0:
                        dss[n - (_NB - 1)].wait()
                    r1 = (n + 1) % _NB
                    dgs[n + 1] = pltpu.async_copy(x_hbm.at[sidx[n + 1]],
                                                  rows[r1], gsem[r1])
            for m in range(max(0, L - _NB), L):
                dss[m].wait()

        for j in range(_BLK):
            issue_idx(j, j)
        pltpu.sync_copy(z_hbm, acc.at[pl.ds(row0, rpt)])
        plsc.subcore_barrier()

        def outer(k, carry):
            base = k * _BLK
            for j in range(_BLK):
                wait_idx(j)
            run_block(_BLK)
            for j in range(_BLK):
                @pl.when(base + _BLK + j < nch)
                def _prefetch():
                    issue_idx(base + _BLK + j, j)
            return carry

        lax.fori_loop(0, nblk, outer, 0)
        if tail:
            for j in range(tail):
                wait_idx(j)
            run_block(tail)
        plsc.subcore_barrier()

        @pl.when(c == 0)
        def _w0():
            pltpu.sync_copy(acc.at[pl.ds(row0, rpt)], out0_hbm.at[pl.ds(row0, rpt)])

        @pl.when(c == 1)
        def _w1():
            pltpu.sync_copy(acc.at[pl.ds(row0, rpt)], out1_hbm.at[pl.ds(row0, rpt)])

    return agg(x, src, dst, zeros)


def _tc_mlp(x, p0, p1, Wa, ba, Wb, bb):
    N, D = x.shape
    Np = p0.shape[0]
    H = Wa.shape[1]
    br = 2000
    grid = (N // br,)

    def body(x_ref, p0_ref, p1_ref, wa_ref, ba_ref, wb_ref, bb_ref, o_ref):
        h = x_ref[...] + p0_ref[...] + p1_ref[...]
        h = jnp.dot(h, wa_ref[...], preferred_element_type=jnp.float32) + ba_ref[...]
        h = jnp.maximum(h, 0.0)
        h = jnp.dot(h, wb_ref[...], preferred_element_type=jnp.float32) + bb_ref[...]
        o_ref[...] = jnp.maximum(h, 0.0)

    return pl.pallas_call(
        body,
        grid=grid,
        in_specs=[
            pl.BlockSpec((br, D), lambda i: (i, 0)),
            pl.BlockSpec((br, D), lambda i: (i, 0)),
            pl.BlockSpec((br, D), lambda i: (i, 0)),
            pl.BlockSpec((D, H), lambda i: (0, 0)),
            pl.BlockSpec((1, H), lambda i: (0, 0)),
            pl.BlockSpec((H, H), lambda i: (0, 0)),
            pl.BlockSpec((1, H), lambda i: (0, 0)),
        ],
        out_specs=pl.BlockSpec((br, H), lambda i: (i, 0)),
        out_shape=jax.ShapeDtypeStruct((N, H), jnp.float32),
    )(x, p0, p1, Wa, ba.reshape(1, -1), Wb, bb.reshape(1, -1))


def kernel(x, edge_index, W1a, b1a, W1b, b1b, W2a, b2a, W2b, b2b):
    ei = edge_index.astype(jnp.int32)
    src, dst = ei[0], ei[1]
    N, D = x.shape
    npad = ((N + 8 * _NS - 1) // (8 * _NS)) * (8 * _NS)
    zeros = jnp.zeros((npad // _NS, D), jnp.float32)
    p0, p1 = _sc_aggregate(x, src, dst, zeros)
    h = _tc_mlp(x, p0, p1, W1a, b1a, W1b, b1b)
    q0, q1 = _sc_aggregate(h, src, dst, zeros)
    return _tc_mlp(h, q0, q1, W2a, b2a, W2b, b2b)


# speedup vs baseline: 8.7182x; 1.0070x over previous
import functools

import jax
import jax.numpy as jnp
from jax import lax
from jax.experimental import pallas as pl
from jax.experimental.pallas import tpu as pltpu
from jax.experimental.pallas import tpu_sc as plsc

_NC, _NS = 2, 16
_NW = _NC * _NS
_CH = 80
_NB = 4
_BLK = 6


def _sc_aggregate(x, src, dst, zeros):
    N, D = x.shape
    E = src.shape[0]
    ch = _CH
    epw = E // _NW
    nch = epw // ch
    nblk = nch // _BLK
    tail = nch - nblk * _BLK
    npad = ((N + 8 * _NS - 1) // (8 * _NS)) * (8 * _NS)
    rpt = npad // _NS
    assert zeros.shape[0] == rpt
    mesh = plsc.VectorSubcoreMesh(core_axis_name="c", subcore_axis_name="s")

    @functools.partial(
        pl.kernel,
        out_type=[jax.ShapeDtypeStruct((npad, D), jnp.float32),
                  jax.ShapeDtypeStruct((npad, D), jnp.float32)],
        mesh=mesh,
        scratch_types=[
            [pltpu.VMEM((ch,), jnp.int32) for _ in range(_BLK)],
            [pltpu.VMEM((ch,), jnp.int32) for _ in range(_BLK)],
            [pltpu.VMEM((ch, D), jnp.float32) for _ in range(_NB)],
            pltpu.VMEM_SHARED((npad, D), jnp.float32),
            [pltpu.SemaphoreType.DMA for _ in range(_NB)],
            [pltpu.SemaphoreType.DMA for _ in range(_NB)],
            [pltpu.SemaphoreType.DMA for _ in range(_BLK)],
        ],
    )
    def agg(x_hbm, src_hbm, dst_hbm, z_hbm, out0_hbm, out1_hbm, sidx, didx,
            rows, acc, gsem, ssem, isem):
        c = lax.axis_index("c")
        s = lax.axis_index("s")
        w = c * _NS + s
        row0 = s * rpt
        base_e = w * epw

        def issue_idx(g, i):
            off = base_e + g * ch
            pltpu.async_copy(src_hbm.at[pl.ds(off, ch)], sidx[i], isem[i])
            pltpu.async_copy(dst_hbm.at[pl.ds(off, ch)], didx[i], isem[i])

        def wait_idx(i):
            pltpu.make_async_copy(src_hbm.at[pl.ds(0, ch)], sidx[i], isem[i]).wait()
            pltpu.make_async_copy(dst_hbm.at[pl.ds(0, ch)], didx[i], isem[i]).wait()

        def run_block(L):
            dgs = [None] * L
            dss = [None] * L
            dgs[0] = pltpu.async_copy(x_hbm.at[sidx[0]], rows[0], gsem[0])
            for n in range(L):
                r = n % _NB
                dgs[n].wait()
                dss[n] = pltpu.async_copy(rows[r], acc.at[didx[n]], ssem[r],
                                          add=True)
                if n + 1 < L:
                    if n - (_NB - 1) >= 0:
                        dss[n - (_NB - 1)].wait()
                    r1 = (n + 1) % _NB
                    dgs[n + 1] = pltpu.async_copy(x_hbm.at[sidx[n + 1]],
                                                  rows[r1], gsem[r1])
            for m in range(max(0, L - _NB), L):
                dss[m].wait()

        for j in range(_BLK):
            issue_idx(j, j)
        pltpu.sync_copy(z_hbm, acc.at[pl.ds(row0, rpt)])
        plsc.subcore_barrier()

        def outer(k, carry):
            base = k * _BLK
            for j in range(_BLK):
                wait_idx(j)
            run_block(_BLK)
            for j in range(_BLK):
                @pl.when(base + _BLK + j < nch)
                def _prefetch():
                    issue_idx(base + _BLK + j, j)
            return carry

        lax.fori_loop(0, nblk, outer, 0)
        if tail:
            for j in range(tail):
                wait_idx(j)
            run_block(tail)
        plsc.subcore_barrier()

        @pl.when(c == 0)
        def _w0():
            pltpu.sync_copy(acc.at[pl.ds(row0, rpt)], out0_hbm.at[pl.ds(row0, rpt)])

        @pl.when(c == 1)
        def _w1():
            pltpu.sync_copy(acc.at[pl.ds(row0, rpt)], out1_hbm.at[pl.ds(row0, rpt)])

    return agg(x, src, dst, zeros)


def _tc_mlp(x, p0, p1, Wa, ba, Wb, bb):
    N, D = x.shape
    Np = p0.shape[0]
    H = Wa.shape[1]
    br = 5000
    grid = (N // br,)

    def body(x_ref, p0_ref, p1_ref, wa_ref, ba_ref, wb_ref, bb_ref, o_ref):
        h = x_ref[...] + p0_ref[...] + p1_ref[...]
        h = jnp.dot(h, wa_ref[...], preferred_element_type=jnp.float32) + ba_ref[...]
        h = jnp.maximum(h, 0.0)
        h = jnp.dot(h, wb_ref[...], preferred_element_type=jnp.float32) + bb_ref[...]
        o_ref[...] = jnp.maximum(h, 0.0)

    return pl.pallas_call(
        body,
        grid=grid,
        in_specs=[
            pl.BlockSpec((br, D), lambda i: (i, 0)),
            pl.BlockSpec((br, D), lambda i: (i, 0)),
            pl.BlockSpec((br, D), lambda i: (i, 0)),
            pl.BlockSpec((D, H), lambda i: (0, 0)),
            pl.BlockSpec((1, H), lambda i: (0, 0)),
            pl.BlockSpec((H, H), lambda i: (0, 0)),
            pl.BlockSpec((1, H), lambda i: (0, 0)),
        ],
        out_specs=pl.BlockSpec((br, H), lambda i: (i, 0)),
        out_shape=jax.ShapeDtypeStruct((N, H), jnp.float32),
    )(x, p0, p1, Wa, ba.reshape(1, -1), Wb, bb.reshape(1, -1))


def kernel(x, edge_index, W1a, b1a, W1b, b1b, W2a, b2a, W2b, b2b):
    ei = edge_index.astype(jnp.int32)
    src, dst = ei[0], ei[1]
    N, D = x.shape
    npad = ((N + 8 * _NS - 1) // (8 * _NS)) * (8 * _NS)
    zeros = jnp.zeros((npad // _NS, D), jnp.float32)
    p0, p1 = _sc_aggregate(x, src, dst, zeros)
    h = _tc_mlp(x, p0, p1, W1a, b1a, W1b, b1b)
    q0, q1 = _sc_aggregate(h, src, dst, zeros)
    return _tc_mlp(h, q0, q1, W2a, b2a, W2b, b2b)


# speedup vs baseline: 8.9664x; 1.0285x over previous
import functools

import jax
import jax.numpy as jnp
from jax import lax
from jax.experimental import pallas as pl
from jax.experimental.pallas import tpu as pltpu
from jax.experimental.pallas import tpu_sc as plsc

_NC, _NS = 2, 16
_NW = _NC * _NS
_CH = 80
_NB = 3
_BLK = 12


def _sc_aggregate(x, src, dst, zeros):
    N, D = x.shape
    E = src.shape[0]
    ch = _CH
    epw = E // _NW
    nch = epw // ch
    nblk = nch // _BLK
    tail = nch - nblk * _BLK
    npad = ((N + 8 * _NS - 1) // (8 * _NS)) * (8 * _NS)
    rpt = npad // _NS
    assert zeros.shape[0] == rpt
    mesh = plsc.VectorSubcoreMesh(core_axis_name="c", subcore_axis_name="s")

    @functools.partial(
        pl.kernel,
        out_type=[jax.ShapeDtypeStruct((npad, D), jnp.float32),
                  jax.ShapeDtypeStruct((npad, D), jnp.float32)],
        mesh=mesh,
        scratch_types=[
            [pltpu.VMEM((ch,), jnp.int32) for _ in range(_BLK)],
            [pltpu.VMEM((ch,), jnp.int32) for _ in range(_BLK)],
            [pltpu.VMEM((ch, D), jnp.float32) for _ in range(_NB)],
            pltpu.VMEM_SHARED((npad, D), jnp.float32),
            [pltpu.SemaphoreType.DMA for _ in range(_NB)],
            [pltpu.SemaphoreType.DMA for _ in range(_NB)],
            [pltpu.SemaphoreType.DMA for _ in range(_BLK)],
        ],
    )
    def agg(x_hbm, src_hbm, dst_hbm, z_hbm, out0_hbm, out1_hbm, sidx, didx,
            rows, acc, gsem, ssem, isem):
        c = lax.axis_index("c")
        s = lax.axis_index("s")
        w = c * _NS + s
        row0 = s * rpt
        base_e = w * epw

        def issue_idx(g, i):
            off = base_e + g * ch
            pltpu.async_copy(src_hbm.at[pl.ds(off, ch)], sidx[i], isem[i])
            pltpu.async_copy(dst_hbm.at[pl.ds(off, ch)], didx[i], isem[i])

        def wait_idx(i):
            pltpu.make_async_copy(src_hbm.at[pl.ds(0, ch)], sidx[i], isem[i]).wait()
            pltpu.make_async_copy(dst_hbm.at[pl.ds(0, ch)], didx[i], isem[i]).wait()

        def run_block(L):
            dgs = [None] * L
            dss = [None] * L
            dgs[0] = pltpu.async_copy(x_hbm.at[sidx[0]], rows[0], gsem[0])
            for n in range(L):
                r = n % _NB
                dgs[n].wait()
                dss[n] = pltpu.async_copy(rows[r], acc.at[didx[n]], ssem[r],
                                          add=True)
                if n + 1 < L:
                    if n - (_NB - 1) >= 0:
                        dss[n - (_NB - 1)].wait()
                    r1 = (n + 1) % _NB
                    dgs[n + 1] = pltpu.async_copy(x_hbm.at[sidx[n + 1]],
                                                  rows[r1], gsem[r1])
            for m in range(max(0, L - _NB), L):
                dss[m].wait()

        for j in range(_BLK):
            issue_idx(j, j)
        pltpu.sync_copy(z_hbm, acc.at[pl.ds(row0, rpt)])
        plsc.subcore_barrier()

        def outer(k, carry):
            base = k * _BLK
            for j in range(_BLK):
                wait_idx(j)
            run_block(_BLK)
            for j in range(_BLK):
                @pl.when(base + _BLK + j < nch)
                def _prefetch():
                    issue_idx(base + _BLK + j, j)
            return carry

        lax.fori_loop(0, nblk, outer, 0)
        if tail:
            for j in range(tail):
                wait_idx(j)
            run_block(tail)
        plsc.subcore_barrier()

        @pl.when(c == 0)
        def _w0():
            pltpu.sync_copy(acc.at[pl.ds(row0, rpt)], out0_hbm.at[pl.ds(row0, rpt)])

        @pl.when(c == 1)
        def _w1():
            pltpu.sync_copy(acc.at[pl.ds(row0, rpt)], out1_hbm.at[pl.ds(row0, rpt)])

    return agg(x, src, dst, zeros)


def _tc_mlp(x, p0, p1, Wa, ba, Wb, bb):
    N, D = x.shape
    Np = p0.shape[0]
    H = Wa.shape[1]
    br = 5000
    grid = (N // br,)

    def body(x_ref, p0_ref, p1_ref, wa_ref, ba_ref, wb_ref, bb_ref, o_ref):
        h = x_ref[...] + p0_ref[...] + p1_ref[...]
        h = jnp.dot(h, wa_ref[...], preferred_element_type=jnp.float32) + ba_ref[...]
        h = jnp.maximum(h, 0.0)
        h = jnp.dot(h, wb_ref[...], preferred_element_type=jnp.float32) + bb_ref[...]
        o_ref[...] = jnp.maximum(h, 0.0)

    return pl.pallas_call(
        body,
        grid=grid,
        in_specs=[
            pl.BlockSpec((br, D), lambda i: (i, 0)),
            pl.BlockSpec((br, D), lambda i: (i, 0)),
            pl.BlockSpec((br, D), lambda i: (i, 0)),
            pl.BlockSpec((D, H), lambda i: (0, 0)),
            pl.BlockSpec((1, H), lambda i: (0, 0)),
            pl.BlockSpec((H, H), lambda i: (0, 0)),
            pl.BlockSpec((1, H), lambda i: (0, 0)),
        ],
        out_specs=pl.BlockSpec((br, H), lambda i: (i, 0)),
        out_shape=jax.ShapeDtypeStruct((N, H), jnp.float32),
    )(x, p0, p1, Wa, ba.reshape(1, -1), Wb, bb.reshape(1, -1))


def kernel(x, edge_index, W1a, b1a, W1b, b1b, W2a, b2a, W2b, b2b):
    ei = edge_index.astype(jnp.int32)
    src, dst = ei[0], ei[1]
    N, D = x.shape
    npad = ((N + 8 * _NS - 1) // (8 * _NS)) * (8 * _NS)
    zeros = jnp.zeros((npad // _NS, D), jnp.float32)
    p0, p1 = _sc_aggregate(x, src, dst, zeros)
    h = _tc_mlp(x, p0, p1, W1a, b1a, W1b, b1b)
    q0, q1 = _sc_aggregate(h, src, dst, zeros)
    return _tc_mlp(h, q0, q1, W2a, b2a, W2b, b2b)


# speedup vs baseline: 9.1191x; 1.0170x over previous
import functools

import jax
import jax.numpy as jnp
from jax import lax
from jax.experimental import pallas as pl
from jax.experimental.pallas import tpu as pltpu
from jax.experimental.pallas import tpu_sc as plsc

_NC, _NS = 2, 16
_NW = _NC * _NS
_CH = 80
_NB = 3
_BLK = 25


def _sc_aggregate(x, src, dst, zeros):
    N, D = x.shape
    E = src.shape[0]
    ch = _CH
    epw = E // _NW
    nch = epw // ch
    nblk = nch // _BLK
    tail = nch - nblk * _BLK
    npad = ((N + 8 * _NS - 1) // (8 * _NS)) * (8 * _NS)
    rpt = npad // _NS
    assert zeros.shape[0] == rpt
    mesh = plsc.VectorSubcoreMesh(core_axis_name="c", subcore_axis_name="s")

    @functools.partial(
        pl.kernel,
        out_type=[jax.ShapeDtypeStruct((npad, D), jnp.float32),
                  jax.ShapeDtypeStruct((npad, D), jnp.float32)],
        mesh=mesh,
        scratch_types=[
            [pltpu.VMEM((ch,), jnp.int32) for _ in range(_BLK)],
            [pltpu.VMEM((ch,), jnp.int32) for _ in range(_BLK)],
            [pltpu.VMEM((ch, D), jnp.float32) for _ in range(_NB)],
            pltpu.VMEM_SHARED((npad, D), jnp.float32),
            [pltpu.SemaphoreType.DMA for _ in range(_NB)],
            [pltpu.SemaphoreType.DMA for _ in range(_NB)],
            [pltpu.SemaphoreType.DMA for _ in range(_BLK)],
        ],
    )
    def agg(x_hbm, src_hbm, dst_hbm, z_hbm, out0_hbm, out1_hbm, sidx, didx,
            rows, acc, gsem, ssem, isem):
        c = lax.axis_index("c")
        s = lax.axis_index("s")
        w = c * _NS + s
        row0 = s * rpt
        base_e = w * epw

        def issue_idx(g, i):
            off = base_e + g * ch
            pltpu.async_copy(src_hbm.at[pl.ds(off, ch)], sidx[i], isem[i])
            pltpu.async_copy(dst_hbm.at[pl.ds(off, ch)], didx[i], isem[i])

        def wait_idx(i):
            pltpu.make_async_copy(src_hbm.at[pl.ds(0, ch)], sidx[i], isem[i]).wait()
            pltpu.make_async_copy(dst_hbm.at[pl.ds(0, ch)], didx[i], isem[i]).wait()

        def run_block(L):
            dgs = [None] * L
            dss = [None] * L
            dgs[0] = pltpu.async_copy(x_hbm.at[sidx[0]], rows[0], gsem[0])
            for n in range(L):
                r = n % _NB
                dgs[n].wait()
                dss[n] = pltpu.async_copy(rows[r], acc.at[didx[n]], ssem[r],
                                          add=True)
                if n + 1 < L:
                    if n - (_NB - 1) >= 0:
                        dss[n - (_NB - 1)].wait()
                    r1 = (n + 1) % _NB
                    dgs[n + 1] = pltpu.async_copy(x_hbm.at[sidx[n + 1]],
                                                  rows[r1], gsem[r1])
            for m in range(max(0, L - _NB), L):
                dss[m].wait()

        for j in range(_BLK):
            issue_idx(j, j)
        pltpu.sync_copy(z_hbm, acc.at[pl.ds(row0, rpt)])
        plsc.subcore_barrier()

        def outer(k, carry):
            base = k * _BLK
            for j in range(_BLK):
                wait_idx(j)
            run_block(_BLK)
            for j in range(_BLK):
                @pl.when(base + _BLK + j < nch)
                def _prefetch():
                    issue_idx(base + _BLK + j, j)
            return carry

        lax.fori_loop(0, nblk, outer, 0)
        if tail:
            for j in range(tail):
                wait_idx(j)
            run_block(tail)
        plsc.subcore_barrier()

        @pl.when(c == 0)
        def _w0():
            pltpu.sync_copy(acc.at[pl.ds(row0, rpt)], out0_hbm.at[pl.ds(row0, rpt)])

        @pl.when(c == 1)
        def _w1():
            pltpu.sync_copy(acc.at[pl.ds(row0, rpt)], out1_hbm.at[pl.ds(row0, rpt)])

    return agg(x, src, dst, zeros)


def _tc_mlp(x, p0, p1, Wa, ba, Wb, bb):
    N, D = x.shape
    Np = p0.shape[0]
    H = Wa.shape[1]
    br = 5000
    grid = (N // br,)

    def body(x_ref, p0_ref, p1_ref, wa_ref, ba_ref, wb_ref, bb_ref, o_ref):
        h = x_ref[...] + p0_ref[...] + p1_ref[...]
        h = jnp.dot(h, wa_ref[...], preferred_element_type=jnp.float32) + ba_ref[...]
        h = jnp.maximum(h, 0.0)
        h = jnp.dot(h, wb_ref[...], preferred_element_type=jnp.float32) + bb_ref[...]
        o_ref[...] = jnp.maximum(h, 0.0)

    return pl.pallas_call(
        body,
        grid=grid,
        in_specs=[
            pl.BlockSpec((br, D), lambda i: (i, 0)),
            pl.BlockSpec((br, D), lambda i: (i, 0)),
            pl.BlockSpec((br, D), lambda i: (i, 0)),
            pl.BlockSpec((D, H), lambda i: (0, 0)),
            pl.BlockSpec((1, H), lambda i: (0, 0)),
            pl.BlockSpec((H, H), lambda i: (0, 0)),
            pl.BlockSpec((1, H), lambda i: (0, 0)),
        ],
        out_specs=pl.BlockSpec((br, H), lambda i: (i, 0)),
        out_shape=jax.ShapeDtypeStruct((N, H), jnp.float32),
    )(x, p0, p1, Wa, ba.reshape(1, -1), Wb, bb.reshape(1, -1))


def kernel(x, edge_index, W1a, b1a, W1b, b1b, W2a, b2a, W2b, b2b):
    ei = edge_index.astype(jnp.int32)
    src, dst = ei[0], ei[1]
    N, D = x.shape
    npad = ((N + 8 * _NS - 1) // (8 * _NS)) * (8 * _NS)
    zeros = jnp.zeros((npad // _NS, D), jnp.float32)
    p0, p1 = _sc_aggregate(x, src, dst, zeros)
    h = _tc_mlp(x, p0, p1, W1a, b1a, W1b, b1b)
    q0, q1 = _sc_aggregate(h, src, dst, zeros)
    return _tc_mlp(h, q0, q1, W2a, b2a, W2b, b2b)


# speedup vs baseline: 9.3618x; 1.0266x over previous
import functools

import jax
import jax.numpy as jnp
from jax import lax
from jax.experimental import pallas as pl
from jax.experimental.pallas import tpu as pltpu
from jax.experimental.pallas import tpu_sc as plsc

_NC, _NS = 2, 16
_NW = _NC * _NS
_CH = 80
_NB = 3
_NBLK = 2


def _sc_aggregate(x, src, dst, zeros):
    N, D = x.shape
    E = src.shape[0]
    ch = _CH
    epw = E // _NW
    nch = epw // ch
    bs = nch // _NBLK
    sizes = [bs] * (_NBLK - 1) + [nch - bs * (_NBLK - 1)]
    nslot = max(sizes)
    npad = ((N + 8 * _NS - 1) // (8 * _NS)) * (8 * _NS)
    rpt = npad // _NS
    assert zeros.shape[0] == rpt
    mesh = plsc.VectorSubcoreMesh(core_axis_name="c", subcore_axis_name="s")

    @functools.partial(
        pl.kernel,
        out_type=[jax.ShapeDtypeStruct((npad, D), jnp.float32),
                  jax.ShapeDtypeStruct((npad, D), jnp.float32)],
        mesh=mesh,
        scratch_types=[
            [pltpu.VMEM((ch,), jnp.int32) for _ in range(nslot)],
            [pltpu.VMEM((ch,), jnp.int32) for _ in range(nslot)],
            [pltpu.VMEM((ch, D), jnp.float32) for _ in range(_NB)],
            pltpu.VMEM_SHARED((npad, D), jnp.float32),
            [pltpu.SemaphoreType.DMA for _ in range(_NB)],
            [pltpu.SemaphoreType.DMA for _ in range(_NB)],
            pltpu.SemaphoreType.DMA,
        ],
    )
    def agg(x_hbm, src_hbm, dst_hbm, z_hbm, out0_hbm, out1_hbm, sidx, didx,
            rows, acc, gsem, ssem, isem):
        c = lax.axis_index("c")
        s = lax.axis_index("s")
        w = c * _NS + s
        row0 = s * rpt
        base_e = w * epw

        def issue_idx(g, i):
            off = base_e + g * ch
            pltpu.async_copy(src_hbm.at[pl.ds(off, ch)], sidx[i], isem)
            pltpu.async_copy(dst_hbm.at[pl.ds(off, ch)], didx[i], isem)

        def wait_idx(i):
            pltpu.make_async_copy(src_hbm.at[pl.ds(0, ch)], sidx[i], isem).wait()
            pltpu.make_async_copy(dst_hbm.at[pl.ds(0, ch)], didx[i], isem).wait()

        def run_block(L):
            dgs = [None] * L
            dss = [None] * L
            dgs[0] = pltpu.async_copy(x_hbm.at[sidx[0]], rows[0], gsem[0])
            for n in range(L):
                r = n % _NB
                dgs[n].wait()
                dss[n] = pltpu.async_copy(rows[r], acc.at[didx[n]], ssem[r],
                                          add=True)
                if n + 1 < L:
                    if n - (_NB - 1) >= 0:
                        dss[n - (_NB - 1)].wait()
                    r1 = (n + 1) % _NB
                    dgs[n + 1] = pltpu.async_copy(x_hbm.at[sidx[n + 1]],
                                                  rows[r1], gsem[r1])
            for m in range(max(0, L - _NB), L):
                dss[m].wait()

        for j in range(sizes[0]):
            issue_idx(j, j)
        pltpu.sync_copy(z_hbm, acc.at[pl.ds(row0, rpt)])
        plsc.subcore_barrier()

        base = 0
        for blk in range(_NBLK):
            L = sizes[blk]
            for j in range(L):
                wait_idx(j)
            run_block(L)
            base += L
            if blk + 1 < _NBLK:
                for j in range(sizes[blk + 1]):
                    issue_idx(base + j, j)
        plsc.subcore_barrier()

        @pl.when(c == 0)
        def _w0():
            pltpu.sync_copy(acc.at[pl.ds(row0, rpt)], out0_hbm.at[pl.ds(row0, rpt)])

        @pl.when(c == 1)
        def _w1():
            pltpu.sync_copy(acc.at[pl.ds(row0, rpt)], out1_hbm.at[pl.ds(row0, rpt)])

    return agg(x, src, dst, zeros)


def _tc_mlp(x, p0, p1, Wa, ba, Wb, bb):
    N, D = x.shape
    Np = p0.shape[0]
    H = Wa.shape[1]
    br = 5000
    grid = (N // br,)

    def body(x_ref, p0_ref, p1_ref, wa_ref, ba_ref, wb_ref, bb_ref, o_ref):
        h = x_ref[...] + p0_ref[...] + p1_ref[...]
        h = jnp.dot(h, wa_ref[...], preferred_element_type=jnp.float32) + ba_ref[...]
        h = jnp.maximum(h, 0.0)
        h = jnp.dot(h, wb_ref[...], preferred_element_type=jnp.float32) + bb_ref[...]
        o_ref[...] = jnp.maximum(h, 0.0)

    return pl.pallas_call(
        body,
        grid=grid,
        in_specs=[
            pl.BlockSpec((br, D), lambda i: (i, 0)),
            pl.BlockSpec((br, D), lambda i: (i, 0)),
            pl.BlockSpec((br, D), lambda i: (i, 0)),
            pl.BlockSpec((D, H), lambda i: (0, 0)),
            pl.BlockSpec((1, H), lambda i: (0, 0)),
            pl.BlockSpec((H, H), lambda i: (0, 0)),
            pl.BlockSpec((1, H), lambda i: (0, 0)),
        ],
        out_specs=pl.BlockSpec((br, H), lambda i: (i, 0)),
        out_shape=jax.ShapeDtypeStruct((N, H), jnp.float32),
    )(x, p0, p1, Wa, ba.reshape(1, -1), Wb, bb.reshape(1, -1))


def kernel(x, edge_index, W1a, b1a, W1b, b1b, W2a, b2a, W2b, b2b):
    ei = edge_index.astype(jnp.int32)
    src, dst = ei[0], ei[1]
    N, D = x.shape
    npad = ((N + 8 * _NS - 1) // (8 * _NS)) * (8 * _NS)
    zeros = jnp.zeros((npad // _NS, D), jnp.float32)
    p0, p1 = _sc_aggregate(x, src, dst, zeros)
    h = _tc_mlp(x, p0, p1, W1a, b1a, W1b, b1b)
    q0, q1 = _sc_aggregate(h, src, dst, zeros)
    return _tc_mlp(h, q0, q1, W2a, b2a, W2b, b2b)
